# baseline (speedup 1.0000x reference)
import jax
import jax.numpy as jnp
from jax.experimental import pallas as pl


def _mm_kernel(x_ref, w_ref, b_ref, o_ref):
    o_ref[...] = jnp.dot(x_ref[...], w_ref[...],
                         preferred_element_type=jnp.float32) + b_ref[...]


def kernel(input, W, b):
    n, in_f = input.shape
    out_f = W.shape[1]
    B = 2048
    return pl.pallas_call(
        _mm_kernel,
        grid=(n // B,),
        in_specs=[
            pl.BlockSpec((B, in_f), lambda i: (i, 0)),
            pl.BlockSpec((in_f, out_f), lambda i: (0, 0)),
            pl.BlockSpec((1, out_f), lambda i: (0, 0)),
        ],
        out_specs=pl.BlockSpec((B, out_f), lambda i: (i, 0)),
        out_shape=jax.ShapeDtypeStruct((n, out_f), jnp.float32),
    )(input, W, b.reshape(1, out_f))


# speedup vs baseline: 1.1939x; 1.1939x over previous
import jax
import jax.numpy as jnp
from jax.experimental import pallas as pl


def _mm_kernel(x_ref, w_ref, b_ref, o_ref):
    o_ref[...] = jnp.dot(x_ref[...], w_ref[...],
                         preferred_element_type=jnp.float32) + b_ref[...]


def kernel(input, W, b):
    n, in_f = input.shape
    out_f = W.shape[1]
    B = 8192
    return pl.pallas_call(
        _mm_kernel,
        grid=(n // B,),
        in_specs=[
            pl.BlockSpec((B, in_f), lambda i: (i, 0)),
            pl.BlockSpec((in_f, out_f), lambda i: (0, 0)),
            pl.BlockSpec((1, out_f), lambda i: (0, 0)),
        ],
        out_specs=pl.BlockSpec((B, out_f), lambda i: (i, 0)),
        out_shape=jax.ShapeDtypeStruct((n, out_f), jnp.float32),
    )(input, W, b.reshape(1, out_f))


# speedup vs baseline: 1.1953x; 1.0012x over previous
import jax
import jax.numpy as jnp
from jax.experimental import pallas as pl


def _mm_kernel(x_ref, w_ref, b_ref, o_ref):
    x = x_ref[...].astype(jnp.bfloat16)
    w = w_ref[...].astype(jnp.bfloat16)
    o_ref[...] = jnp.dot(x, w, preferred_element_type=jnp.float32) + b_ref[...]


def kernel(input, W, b):
    n, in_f = input.shape
    out_f = W.shape[1]
    B = 8192
    return pl.pallas_call(
        _mm_kernel,
        grid=(n // B,),
        in_specs=[
            pl.BlockSpec((B, in_f), lambda i: (i, 0)),
            pl.BlockSpec((in_f, out_f), lambda i: (0, 0)),
            pl.BlockSpec((1, out_f), lambda i: (0, 0)),
        ],
        out_specs=pl.BlockSpec((B, out_f), lambda i: (i, 0)),
        out_shape=jax.ShapeDtypeStruct((n, out_f), jnp.float32),
    )(input, W, b.reshape(1, out_f))
